# initial kernel scaffold (unmeasured)
import jax
import jax.numpy as jnp
from jax import lax
from jax.experimental import pallas as pl
from jax.experimental.pallas import tpu as pltpu

N = 8
B = 2
SQ = 512
SKV = 4096
HQ = 64
DH = 64
H_LOC = HQ // N
KV_LOC = SKV // N
DM = 768
CH = SQ // N
QBLK = 64

F32 = jnp.float32
BF16 = jnp.bfloat16


def kernel(x, Wq, K_ext, V_ext, Wo):
    def body(x_ref, wq_ref, k_ext_ref, v_ext_ref, wo_ref, out_ref,
             k_all, v_all, kbuf, vbuf, qbuf, mask_ref, partial_ref,
             rs_buf, red_buf,
             k_send_sems, k_recv_sems, v_send_sems, v_recv_sems,
             rs_send_sems, rs_recv_sems, ag_send_sems, ag_recv_sems,
             local_sems):
        my = lax.axis_index("i")

        kv_sends = []
        for d in range(1, N):
            peer = lax.rem(my + d, N)
            k_rdma = pltpu.make_async_remote_copy(
                src_ref=k_ext_ref.at[:, :, pl.ds(peer * H_LOC, H_LOC), :],
                dst_ref=k_all.at[my],
                send_sem=k_send_sems.at[peer],
                recv_sem=k_recv_sems.at[my],
                device_id=(peer,),
                device_id_type=pl.DeviceIdType.MESH,
            )
            k_rdma.start()
            v_rdma = pltpu.make_async_remote_copy(
                src_ref=v_ext_ref.at[:, :, pl.ds(peer * H_LOC, H_LOC), :],
                dst_ref=v_all.at[my],
                send_sem=v_send_sems.at[peer],
                recv_sem=v_recv_sems.at[my],
                device_id=(peer,),
                device_id_type=pl.DeviceIdType.MESH,
            )
            v_rdma.start()
            kv_sends.append(k_rdma)
            kv_sends.append(v_rdma)

        lk = pltpu.make_async_copy(
            k_ext_ref.at[:, :, pl.ds(my * H_LOC, H_LOC), :],
            k_all.at[my], local_sems.at[0])
        lk.start()
        lv = pltpu.make_async_copy(
            v_ext_ref.at[:, :, pl.ds(my * H_LOC, H_LOC), :],
            v_all.at[my], local_sems.at[1])
        lv.start()

        wq = wq_ref[...].astype(BF16)
        for b in range(B):
            q = lax.dot_general(
                x_ref[b].astype(BF16), wq,
                (((1,), (0,)), ((), ())),
                preferred_element_type=F32)
            qbuf[b] = q.reshape(SQ, H_LOC, DH).astype(BF16)

        qb = lax.broadcasted_iota(jnp.int32, (SQ, SKV), 0) // QBLK
        kb = lax.broadcasted_iota(jnp.int32, (SQ, SKV), 1) // QBLK
        allow = (qb == kb) | (kb == 0) | (lax.rem(qb + kb, 3) == 0)
        mask_ref[...] = jnp.where(allow, 0.0, -1e9).astype(BF16)

        lk.wait()
        lv.wait()
        for d in range(1, N):
            src = lax.rem(my - d + N, N)
            pltpu.make_async_remote_copy(
                src_ref=k_ext_ref.at[:, :, pl.ds(0, H_LOC), :],
                dst_ref=k_all.at[src],
                send_sem=k_send_sems.at[src],
                recv_sem=k_recv_sems.at[src],
                device_id=(src,),
                device_id_type=pl.DeviceIdType.MESH,
            ).wait_recv()
            pltpu.make_async_remote_copy(
                src_ref=v_ext_ref.at[:, :, pl.ds(0, H_LOC), :],
                dst_ref=v_all.at[src],
                send_sem=v_send_sems.at[src],
                recv_sem=v_recv_sems.at[src],
                device_id=(src,),
                device_id_type=pl.DeviceIdType.MESH,
            ).wait_recv()
        for rdma in kv_sends:
            rdma.wait_send()

        m0 = jnp.full((B, SQ, H_LOC, 1), -1e30, F32)
        l0 = jnp.zeros((B, SQ, H_LOC, 1), F32)
        a0 = jnp.zeros((B, SQ, H_LOC, DH), F32)

        def slot_step(j, carry):
            m, l, acc = carry
            ck = pltpu.make_async_copy(k_all.at[j], kbuf, local_sems.at[0])
            ck.start()
            cv = pltpu.make_async_copy(v_all.at[j], vbuf, local_sems.at[1])
            cv.start()
            ck.wait()
            cv.wait()
            mask_j = mask_ref[:, pl.ds(j * KV_LOC, KV_LOC)].astype(F32)
            for b in range(B):
                for h in range(H_LOC):
                    q_bh = qbuf[b, :, h, :]
                    k_bh = kbuf[b, :, h, :].astype(BF16)
                    s = lax.dot_general(
                        q_bh, k_bh, (((1,), (1,)), ((), ())),
                        preferred_element_type=F32) * 0.125 + mask_j
                    m_old = m[b, :, h, :]
                    m_new = jnp.maximum(m_old, jnp.max(s, axis=-1, keepdims=True))
                    p = jnp.exp(s - m_new)
                    corr = jnp.exp(m_old - m_new)
                    l_new = l[b, :, h, :] * corr + jnp.sum(p, axis=-1, keepdims=True)
                    v_bh = vbuf[b, :, h, :].astype(BF16)
                    pv = lax.dot_general(
                        p.astype(BF16), v_bh, (((1,), (0,)), ((), ())),
                        preferred_element_type=F32)
                    acc_new = acc[b, :, h, :] * corr + pv
                    m = m.at[b, :, h, :].set(m_new)
                    l = l.at[b, :, h, :].set(l_new)
                    acc = acc.at[b, :, h, :].set(acc_new)
            return m, l, acc

        m, l, acc = lax.fori_loop(0, N, slot_step, (m0, l0, a0))

        wo = wo_ref[...].astype(BF16)
        for b in range(B):
            ctx_b = (acc[b] / l[b]).reshape(SQ, H_LOC * DH).astype(BF16)
            partial_ref[b] = lax.dot_general(
                ctx_b, wo, (((1,), (0,)), ((), ())),
                preferred_element_type=F32)

        rs_sends = []
        for d in range(1, N):
            peer = lax.rem(my + d, N)
            rdma = pltpu.make_async_remote_copy(
                src_ref=partial_ref.at[:, pl.ds(peer * CH, CH), :],
                dst_ref=rs_buf.at[my],
                send_sem=rs_send_sems.at[peer],
                recv_sem=rs_recv_sems.at[my],
                device_id=(peer,),
                device_id_type=pl.DeviceIdType.MESH,
            )
            rdma.start()
            rs_sends.append(rdma)
        red = partial_ref[:, pl.ds(my * CH, CH), :]
        for d in range(1, N):
            src = lax.rem(my - d + N, N)
            pltpu.make_async_remote_copy(
                src_ref=partial_ref.at[:, pl.ds(0, CH), :],
                dst_ref=rs_buf.at[src],
                send_sem=rs_send_sems.at[src],
                recv_sem=rs_recv_sems.at[src],
                device_id=(src,),
                device_id_type=pl.DeviceIdType.MESH,
            ).wait_recv()
            red = red + rs_buf[src]
        red_buf[...] = red
        out_ref[:, pl.ds(my * CH, CH), :] = red

        ag_sends = []
        for d in range(1, N):
            peer = lax.rem(my + d, N)
            rdma = pltpu.make_async_remote_copy(
                src_ref=red_buf,
                dst_ref=out_ref.at[:, pl.ds(my * CH, CH), :],
                send_sem=ag_send_sems.at[peer],
                recv_sem=ag_recv_sems.at[my],
                device_id=(peer,),
                device_id_type=pl.DeviceIdType.MESH,
            )
            rdma.start()
            ag_sends.append(rdma)
        for d in range(1, N):
            src = lax.rem(my - d + N, N)
            pltpu.make_async_remote_copy(
                src_ref=red_buf,
                dst_ref=out_ref.at[:, pl.ds(src * CH, CH), :],
                send_sem=ag_send_sems.at[src],
                recv_sem=ag_recv_sems.at[src],
                device_id=(src,),
                device_id_type=pl.DeviceIdType.MESH,
            ).wait_recv()
        for rdma in rs_sends + ag_sends:
            rdma.wait_send()

    vmem = pltpu.MemorySpace.VMEM
    hbm = pltpu.MemorySpace.HBM
    return pl.pallas_call(
        body,
        out_shape=jax.ShapeDtypeStruct((B, SQ, DM), F32),
        in_specs=[
            pl.BlockSpec(memory_space=vmem),
            pl.BlockSpec(memory_space=vmem),
            pl.BlockSpec(memory_space=hbm),
            pl.BlockSpec(memory_space=hbm),
            pl.BlockSpec(memory_space=vmem),
        ],
        out_specs=pl.BlockSpec(memory_space=vmem),
        scratch_shapes=[
            hbm((N, B, KV_LOC, H_LOC, DH), F32),
            hbm((N, B, KV_LOC, H_LOC, DH), F32),
            vmem((B, KV_LOC, H_LOC, DH), F32),
            vmem((B, KV_LOC, H_LOC, DH), F32),
            vmem((B, SQ, H_LOC, DH), BF16),
            vmem((SQ, SKV), BF16),
            vmem((B, SQ, DM), F32),
            vmem((N, B, CH, DM), F32),
            vmem((B, CH, DM), F32),
            pltpu.SemaphoreType.DMA((N,)),
            pltpu.SemaphoreType.DMA((N,)),
            pltpu.SemaphoreType.DMA((N,)),
            pltpu.SemaphoreType.DMA((N,)),
            pltpu.SemaphoreType.DMA((N,)),
            pltpu.SemaphoreType.DMA((N,)),
            pltpu.SemaphoreType.DMA((N,)),
            pltpu.SemaphoreType.DMA((N,)),
            pltpu.SemaphoreType.DMA((2,)),
        ],
    )(x, Wq, K_ext, V_ext, Wo)


# baseline (device time: 856288 ns/iter reference)
import jax
import jax.numpy as jnp
from jax import lax
from jax.experimental import pallas as pl
from jax.experimental.pallas import tpu as pltpu

N = 8
B = 2
SQ = 512
SKV = 4096
HQ = 64
DH = 64
H_LOC = HQ // N
KV_LOC = SKV // N
DM = 768
CH = SQ // N
QBLK = 64

F32 = jnp.float32
BF16 = jnp.bfloat16


def kernel(x, Wq, K_ext, V_ext, Wo):
    def body(x_ref, wq_ref, k_ext_ref, v_ext_ref, wo_ref,
             out_ref, k_all, v_all,
             kbuf, vbuf, qbuf, mask_ref, partial_ref,
             rs_buf, red_buf,
             k_send_sems, k_recv_sems, v_send_sems, v_recv_sems,
             rs_send_sems, rs_recv_sems, ag_send_sems, ag_recv_sems,
             local_sems):
        my = lax.axis_index("i")

        kv_sends = []
        for d in range(1, N):
            peer = lax.rem(my + d, N)
            k_rdma = pltpu.make_async_remote_copy(
                src_ref=k_ext_ref.at[:, :, pl.ds(peer * H_LOC, H_LOC), :],
                dst_ref=k_all.at[my],
                send_sem=k_send_sems.at[peer],
                recv_sem=k_recv_sems.at[my],
                device_id=(peer,),
                device_id_type=pl.DeviceIdType.MESH,
            )
            k_rdma.start()
            v_rdma = pltpu.make_async_remote_copy(
                src_ref=v_ext_ref.at[:, :, pl.ds(peer * H_LOC, H_LOC), :],
                dst_ref=v_all.at[my],
                send_sem=v_send_sems.at[peer],
                recv_sem=v_recv_sems.at[my],
                device_id=(peer,),
                device_id_type=pl.DeviceIdType.MESH,
            )
            v_rdma.start()
            kv_sends.append(k_rdma)
            kv_sends.append(v_rdma)

        lk = pltpu.make_async_copy(
            k_ext_ref.at[:, :, pl.ds(my * H_LOC, H_LOC), :],
            k_all.at[my], local_sems.at[0])
        lk.start()
        lv = pltpu.make_async_copy(
            v_ext_ref.at[:, :, pl.ds(my * H_LOC, H_LOC), :],
            v_all.at[my], local_sems.at[1])
        lv.start()

        wq = wq_ref[...].astype(BF16)
        for b in range(B):
            q = lax.dot_general(
                x_ref[b].astype(BF16), wq,
                (((1,), (0,)), ((), ())),
                preferred_element_type=F32)
            qbuf[b] = q.reshape(SQ, H_LOC, DH).astype(BF16)

        qb = lax.broadcasted_iota(jnp.int32, (SQ, SKV), 0) // QBLK
        kb = lax.broadcasted_iota(jnp.int32, (SQ, SKV), 1) // QBLK
        allow = (qb == kb) | (kb == 0) | (lax.rem(qb + kb, 3) == 0)
        mask_ref[...] = jnp.where(allow, 0.0, -1e9).astype(BF16)

        lk.wait()
        lv.wait()
        for d in range(1, N):
            src = lax.rem(my - d + N, N)
            pltpu.make_async_remote_copy(
                src_ref=k_ext_ref.at[:, :, pl.ds(0, H_LOC), :],
                dst_ref=k_all.at[src],
                send_sem=k_send_sems.at[src],
                recv_sem=k_recv_sems.at[src],
                device_id=(src,),
                device_id_type=pl.DeviceIdType.MESH,
            ).wait_recv()
            pltpu.make_async_remote_copy(
                src_ref=v_ext_ref.at[:, :, pl.ds(0, H_LOC), :],
                dst_ref=v_all.at[src],
                send_sem=v_send_sems.at[src],
                recv_sem=v_recv_sems.at[src],
                device_id=(src,),
                device_id_type=pl.DeviceIdType.MESH,
            ).wait_recv()
        for rdma in kv_sends:
            rdma.wait_send()

        BH = B * H_LOC
        m0 = tuple(jnp.full((SQ, 1), -1e30, F32) for _ in range(BH))
        l0 = tuple(jnp.zeros((SQ, 1), F32) for _ in range(BH))
        a0 = tuple(jnp.zeros((SQ, DH), F32) for _ in range(BH))

        def slot_step(j, carry):
            ms, ls, accs = carry
            ck = pltpu.make_async_copy(k_all.at[j], kbuf, local_sems.at[0])
            ck.start()
            cv = pltpu.make_async_copy(v_all.at[j], vbuf, local_sems.at[1])
            cv.start()
            ck.wait()
            cv.wait()
            mask_j = mask_ref[:, pl.ds(j * KV_LOC, KV_LOC)].astype(F32)
            ms_n, ls_n, accs_n = [], [], []
            for b in range(B):
                for h in range(H_LOC):
                    t = b * H_LOC + h
                    q_bh = qbuf[b, :, h, :]
                    k_bh = kbuf[b, :, h, :].astype(BF16)
                    s = lax.dot_general(
                        q_bh, k_bh, (((1,), (1,)), ((), ())),
                        preferred_element_type=F32) * 0.125 + mask_j
                    m_old = ms[t]
                    m_new = jnp.maximum(m_old, jnp.max(s, axis=-1, keepdims=True))
                    p = jnp.exp(s - m_new)
                    corr = jnp.exp(m_old - m_new)
                    l_new = ls[t] * corr + jnp.sum(p, axis=-1, keepdims=True)
                    v_bh = vbuf[b, :, h, :].astype(BF16)
                    pv = lax.dot_general(
                        p.astype(BF16), v_bh, (((1,), (0,)), ((), ())),
                        preferred_element_type=F32)
                    acc_new = accs[t] * corr + pv
                    ms_n.append(m_new)
                    ls_n.append(l_new)
                    accs_n.append(acc_new)
            return tuple(ms_n), tuple(ls_n), tuple(accs_n)

        ms, ls, accs = lax.fori_loop(0, N, slot_step, (m0, l0, a0))

        wo = wo_ref[...].astype(BF16)
        for b in range(B):
            ctx_b = jnp.concatenate(
                [accs[b * H_LOC + h] / ls[b * H_LOC + h]
                 for h in range(H_LOC)], axis=1).astype(BF16)
            partial_ref[b] = lax.dot_general(
                ctx_b, wo, (((1,), (0,)), ((), ())),
                preferred_element_type=F32)

        rs_sends = []
        for d in range(1, N):
            peer = lax.rem(my + d, N)
            rdma = pltpu.make_async_remote_copy(
                src_ref=partial_ref.at[:, pl.ds(peer * CH, CH), :],
                dst_ref=rs_buf.at[my],
                send_sem=rs_send_sems.at[peer],
                recv_sem=rs_recv_sems.at[my],
                device_id=(peer,),
                device_id_type=pl.DeviceIdType.MESH,
            )
            rdma.start()
            rs_sends.append(rdma)
        red = partial_ref[:, pl.ds(my * CH, CH), :]
        for d in range(1, N):
            src = lax.rem(my - d + N, N)
            pltpu.make_async_remote_copy(
                src_ref=partial_ref.at[:, pl.ds(0, CH), :],
                dst_ref=rs_buf.at[src],
                send_sem=rs_send_sems.at[src],
                recv_sem=rs_recv_sems.at[src],
                device_id=(src,),
                device_id_type=pl.DeviceIdType.MESH,
            ).wait_recv()
            red = red + rs_buf[src]
        red_buf[...] = red
        out_ref[:, pl.ds(my * CH, CH), :] = red

        ag_sends = []
        for d in range(1, N):
            peer = lax.rem(my + d, N)
            rdma = pltpu.make_async_remote_copy(
                src_ref=red_buf,
                dst_ref=out_ref.at[:, pl.ds(my * CH, CH), :],
                send_sem=ag_send_sems.at[peer],
                recv_sem=ag_recv_sems.at[my],
                device_id=(peer,),
                device_id_type=pl.DeviceIdType.MESH,
            )
            rdma.start()
            ag_sends.append(rdma)
        for d in range(1, N):
            src = lax.rem(my - d + N, N)
            pltpu.make_async_remote_copy(
                src_ref=red_buf,
                dst_ref=out_ref.at[:, pl.ds(src * CH, CH), :],
                send_sem=ag_send_sems.at[src],
                recv_sem=ag_recv_sems.at[src],
                device_id=(src,),
                device_id_type=pl.DeviceIdType.MESH,
            ).wait_recv()
        for rdma in rs_sends + ag_sends:
            rdma.wait_send()

    vmem = pltpu.MemorySpace.VMEM
    hbm = pltpu.MemorySpace.HBM
    out = pl.pallas_call(
        body,
        out_shape=[
            jax.ShapeDtypeStruct((B, SQ, DM), F32),
            jax.ShapeDtypeStruct((N, B, KV_LOC, H_LOC, DH), F32),
            jax.ShapeDtypeStruct((N, B, KV_LOC, H_LOC, DH), F32),
        ],
        in_specs=[
            pl.BlockSpec(memory_space=vmem),
            pl.BlockSpec(memory_space=vmem),
            pl.BlockSpec(memory_space=hbm),
            pl.BlockSpec(memory_space=hbm),
            pl.BlockSpec(memory_space=vmem),
        ],
        out_specs=[
            pl.BlockSpec(memory_space=vmem),
            pl.BlockSpec(memory_space=hbm),
            pl.BlockSpec(memory_space=hbm),
        ],
        scratch_shapes=[
            vmem((B, KV_LOC, H_LOC, DH), F32),
            vmem((B, KV_LOC, H_LOC, DH), F32),
            vmem((B, SQ, H_LOC, DH), BF16),
            vmem((SQ, SKV), BF16),
            vmem((B, SQ, DM), F32),
            vmem((N, B, CH, DM), F32),
            vmem((B, CH, DM), F32),
            pltpu.SemaphoreType.DMA((N,)),
            pltpu.SemaphoreType.DMA((N,)),
            pltpu.SemaphoreType.DMA((N,)),
            pltpu.SemaphoreType.DMA((N,)),
            pltpu.SemaphoreType.DMA((N,)),
            pltpu.SemaphoreType.DMA((N,)),
            pltpu.SemaphoreType.DMA((N,)),
            pltpu.SemaphoreType.DMA((N,)),
            pltpu.SemaphoreType.DMA((2,)),
        ],
        compiler_params=pltpu.CompilerParams(
            vmem_limit_bytes=100 * 1024 * 1024,
        ),
    )(x, Wq, K_ext, V_ext, Wo)
    return out[0]


# device time: 375490 ns/iter; 2.2805x vs baseline; 2.2805x over previous
import jax
import jax.numpy as jnp
from jax import lax
from jax.experimental import pallas as pl
from jax.experimental.pallas import tpu as pltpu

N = 8
B = 2
SQ = 512
SKV = 4096
HQ = 64
DH = 64
H_LOC = HQ // N
KV_LOC = SKV // N
HD = H_LOC * DH
DM = 768
CH = SQ // N
QBLK = 64
HCH = 8

F32 = jnp.float32
BF16 = jnp.bfloat16


def kernel(x, Wq, K_ext, V_ext, Wo):
    def body(x_ref, wq_ref, k_ext_ref, v_ext_ref, wo_ref,
             out_ref, krx, vrx,
             kcast, vcast, kbuf, vbuf, tmp, qbuf,
             m_ref, l_ref, acc_ref, partial_ref, rs_buf, red_buf,
             k_send_sems, k_recv_sems, v_send_sems, v_recv_sems,
             rs_send_sems, rs_recv_sems, ag_send_sems, ag_recv_sems,
             local_sems):
        my = lax.axis_index("i")

        for src_ref, cast_ref in ((k_ext_ref, kcast), (v_ext_ref, vcast)):
            for b in range(B):
                for c in range(HQ // HCH):
                    cp = pltpu.make_async_copy(
                        src_ref.at[b, :, pl.ds(c * HCH, HCH), :],
                        tmp, local_sems.at[0])
                    cp.start()
                    cp.wait()
                    cast_ref[b, :, pl.ds(c * HCH * DH, HCH * DH)] = (
                        tmp[...].astype(BF16).reshape(KV_LOC, HCH * DH))

        kv_sends = []
        for d in range(1, N):
            peer = lax.rem(my + d, N)
            k_rdma = pltpu.make_async_remote_copy(
                src_ref=kcast.at[:, :, pl.ds(peer * HD, HD)],
                dst_ref=krx.at[my],
                send_sem=k_send_sems.at[peer],
                recv_sem=k_recv_sems.at[my],
                device_id=(peer,),
                device_id_type=pl.DeviceIdType.MESH,
            )
            k_rdma.start()
            v_rdma = pltpu.make_async_remote_copy(
                src_ref=vcast.at[:, :, pl.ds(peer * HD, HD)],
                dst_ref=vrx.at[my],
                send_sem=v_send_sems.at[peer],
                recv_sem=v_recv_sems.at[my],
                device_id=(peer,),
                device_id_type=pl.DeviceIdType.MESH,
            )
            v_rdma.start()
            kv_sends.append(k_rdma)
            kv_sends.append(v_rdma)

        lk = pltpu.make_async_copy(
            kcast.at[:, :, pl.ds(my * HD, HD)], kbuf, local_sems.at[0])
        lk.start()
        lv = pltpu.make_async_copy(
            vcast.at[:, :, pl.ds(my * HD, HD)], vbuf, local_sems.at[1])
        lv.start()

        wq = wq_ref[...].astype(BF16)
        for b in range(B):
            q = lax.dot_general(
                x_ref[b].astype(BF16), wq,
                (((1,), (0,)), ((), ())),
                preferred_element_type=F32)
            qbuf[b] = (q * 0.125).astype(BF16)

        qb_iota = lax.broadcasted_iota(jnp.int32, (SQ, KV_LOC), 0) // QBLK
        kb_loc = lax.broadcasted_iota(jnp.int32, (SQ, KV_LOC), 1) // QBLK

        def slot_update(src, first):
            kb = kb_loc + src * (KV_LOC // QBLK)
            allow = (qb_iota == kb) | (kb == 0) | (lax.rem(qb_iota + kb, 3) == 0)
            mask_j = jnp.where(allow, 0.0, -1e9).astype(F32)
            for b in range(B):
                for h in range(H_LOC):
                    q_bh = qbuf[b, :, pl.ds(h * DH, DH)]
                    k_bh = kbuf[b, :, pl.ds(h * DH, DH)]
                    s = lax.dot_general(
                        q_bh, k_bh, (((1,), (1,)), ((), ())),
                        preferred_element_type=F32) + mask_j
                    m_new = jnp.max(s, axis=-1, keepdims=True)
                    if not first:
                        m_old = m_ref[b, :, pl.ds(h, 1)]
                        m_new = jnp.maximum(m_old, m_new)
                    p = jnp.exp(s - m_new)
                    psum = jnp.sum(p, axis=-1, keepdims=True)
                    v_bh = vbuf[b, :, pl.ds(h * DH, DH)]
                    pv = lax.dot_general(
                        p.astype(BF16), v_bh, (((1,), (0,)), ((), ())),
                        preferred_element_type=F32)
                    if first:
                        l_new, acc_new = psum, pv
                    else:
                        corr = jnp.exp(m_old - m_new)
                        l_new = l_ref[b, :, pl.ds(h, 1)] * corr + psum
                        acc_new = acc_ref[b, :, pl.ds(h * DH, DH)] * corr + pv
                    m_ref[b, :, pl.ds(h, 1)] = m_new
                    l_ref[b, :, pl.ds(h, 1)] = l_new
                    acc_ref[b, :, pl.ds(h * DH, DH)] = acc_new

        lk.wait()
        lv.wait()
        slot_update(my, first=True)

        def slot_step(d, carry):
            src = lax.rem(my - d + N, N)
            pltpu.make_async_remote_copy(
                src_ref=kcast.at[:, :, pl.ds(0, HD)],
                dst_ref=krx.at[src],
                send_sem=k_send_sems.at[src],
                recv_sem=k_recv_sems.at[src],
                device_id=(src,),
                device_id_type=pl.DeviceIdType.MESH,
            ).wait_recv()
            pltpu.make_async_remote_copy(
                src_ref=vcast.at[:, :, pl.ds(0, HD)],
                dst_ref=vrx.at[src],
                send_sem=v_send_sems.at[src],
                recv_sem=v_recv_sems.at[src],
                device_id=(src,),
                device_id_type=pl.DeviceIdType.MESH,
            ).wait_recv()
            fk = pltpu.make_async_copy(krx.at[src], kbuf, local_sems.at[0])
            fk.start()
            fv = pltpu.make_async_copy(vrx.at[src], vbuf, local_sems.at[1])
            fv.start()
            fk.wait()
            fv.wait()
            slot_update(src, first=False)
            return carry

        lax.fori_loop(1, N, slot_step, jnp.int32(0))
        for rdma in kv_sends:
            rdma.wait_send()

        wo = wo_ref[...].astype(BF16)
        for b in range(B):
            acc_b = acc_ref[b].reshape(SQ, H_LOC, DH)
            l_b = l_ref[b].reshape(SQ, H_LOC, 1)
            ctx_b = (acc_b / l_b).reshape(SQ, HD).astype(BF16)
            partial_ref[b] = lax.dot_general(
                ctx_b, wo, (((1,), (0,)), ((), ())),
                preferred_element_type=F32)

        rs_sends = []
        for d in range(1, N):
            peer = lax.rem(my + d, N)
            rdma = pltpu.make_async_remote_copy(
                src_ref=partial_ref.at[:, pl.ds(peer * CH, CH), :],
                dst_ref=rs_buf.at[my],
                send_sem=rs_send_sems.at[peer],
                recv_sem=rs_recv_sems.at[my],
                device_id=(peer,),
                device_id_type=pl.DeviceIdType.MESH,
            )
            rdma.start()
            rs_sends.append(rdma)
        red = partial_ref[:, pl.ds(my * CH, CH), :]
        for d in range(1, N):
            src = lax.rem(my - d + N, N)
            pltpu.make_async_remote_copy(
                src_ref=partial_ref.at[:, pl.ds(0, CH), :],
                dst_ref=rs_buf.at[src],
                send_sem=rs_send_sems.at[src],
                recv_sem=rs_recv_sems.at[src],
                device_id=(src,),
                device_id_type=pl.DeviceIdType.MESH,
            ).wait_recv()
            red = red + rs_buf[src]
        red_buf[...] = red
        out_ref[:, pl.ds(my * CH, CH), :] = red

        ag_sends = []
        for d in range(1, N):
            peer = lax.rem(my + d, N)
            rdma = pltpu.make_async_remote_copy(
                src_ref=red_buf,
                dst_ref=out_ref.at[:, pl.ds(my * CH, CH), :],
                send_sem=ag_send_sems.at[peer],
                recv_sem=ag_recv_sems.at[my],
                device_id=(peer,),
                device_id_type=pl.DeviceIdType.MESH,
            )
            rdma.start()
            ag_sends.append(rdma)
        for d in range(1, N):
            src = lax.rem(my - d + N, N)
            pltpu.make_async_remote_copy(
                src_ref=red_buf,
                dst_ref=out_ref.at[:, pl.ds(src * CH, CH), :],
                send_sem=ag_send_sems.at[src],
                recv_sem=ag_recv_sems.at[src],
                device_id=(src,),
                device_id_type=pl.DeviceIdType.MESH,
            ).wait_recv()
        for rdma in rs_sends + ag_sends:
            rdma.wait_send()

    vmem = pltpu.MemorySpace.VMEM
    hbm = pltpu.MemorySpace.HBM
    out = pl.pallas_call(
        body,
        out_shape=[
            jax.ShapeDtypeStruct((B, SQ, DM), F32),
            jax.ShapeDtypeStruct((N, B, KV_LOC, HD), BF16),
            jax.ShapeDtypeStruct((N, B, KV_LOC, HD), BF16),
        ],
        in_specs=[
            pl.BlockSpec(memory_space=vmem),
            pl.BlockSpec(memory_space=vmem),
            pl.BlockSpec(memory_space=hbm),
            pl.BlockSpec(memory_space=hbm),
            pl.BlockSpec(memory_space=vmem),
        ],
        out_specs=[
            pl.BlockSpec(memory_space=vmem),
            pl.BlockSpec(memory_space=hbm),
            pl.BlockSpec(memory_space=hbm),
        ],
        scratch_shapes=[
            vmem((B, KV_LOC, HQ * DH), BF16),
            vmem((B, KV_LOC, HQ * DH), BF16),
            vmem((B, KV_LOC, HD), BF16),
            vmem((B, KV_LOC, HD), BF16),
            vmem((KV_LOC, HCH, DH), F32),
            vmem((B, SQ, HD), BF16),
            vmem((B, SQ, H_LOC), F32),
            vmem((B, SQ, H_LOC), F32),
            vmem((B, SQ, HD), F32),
            vmem((B, SQ, DM), F32),
            vmem((N, B, CH, DM), F32),
            vmem((B, CH, DM), F32),
            pltpu.SemaphoreType.DMA((N,)),
            pltpu.SemaphoreType.DMA((N,)),
            pltpu.SemaphoreType.DMA((N,)),
            pltpu.SemaphoreType.DMA((N,)),
            pltpu.SemaphoreType.DMA((N,)),
            pltpu.SemaphoreType.DMA((N,)),
            pltpu.SemaphoreType.DMA((N,)),
            pltpu.SemaphoreType.DMA((N,)),
            pltpu.SemaphoreType.DMA((2,)),
        ],
        compiler_params=pltpu.CompilerParams(
            vmem_limit_bytes=100 * 1024 * 1024,
        ),
    )(x, Wq, K_ext, V_ext, Wo)
    return out[0]


# device time: 326712 ns/iter; 2.6209x vs baseline; 1.1493x over previous
import jax
import jax.numpy as jnp
from jax import lax
from jax.experimental import pallas as pl
from jax.experimental.pallas import tpu as pltpu

N = 8
B = 2
SQ = 512
SKV = 4096
HQ = 64
DH = 64
H_LOC = HQ // N
KV_LOC = SKV // N
HD = H_LOC * DH
DM = 768
CH = SQ // N
QBLK = 64
HCH = 8

F32 = jnp.float32
BF16 = jnp.bfloat16


def kernel(x, Wq, K_ext, V_ext, Wo):
    def body(x_ref, wq_ref, k_ext_ref, v_ext_ref, wo_ref,
             out_ref, krx, vrx,
             kcast, vcast, kbuf, vbuf, tmp, qbuf,
             m_ref, l_ref, acc_ref, partial_ref, rs_buf, red_buf, agbuf,
             k_send_sems, k_recv_sems, v_send_sems, v_recv_sems,
             rs_send_sems, rs_recv_sems, ag_send_sems, ag_recv_sems,
             local_sems):
        my = lax.axis_index("i")

        kv_sends = []
        for d in range(1, N + 1):
            c = lax.rem(my + d, N)
            for src_ref, cast_ref in ((k_ext_ref, kcast), (v_ext_ref, vcast)):
                for b in range(B):
                    cp = pltpu.make_async_copy(
                        src_ref.at[b, :, pl.ds(c * HCH, HCH), :],
                        tmp, local_sems.at[0])
                    cp.start()
                    cp.wait()
                    cast_ref[c, b] = tmp[...].astype(BF16).reshape(KV_LOC, HD)
            if d < N:
                k_rdma = pltpu.make_async_remote_copy(
                    src_ref=kcast.at[c],
                    dst_ref=krx.at[my],
                    send_sem=k_send_sems.at[c],
                    recv_sem=k_recv_sems.at[my],
                    device_id=(c,),
                    device_id_type=pl.DeviceIdType.MESH,
                )
                k_rdma.start()
                v_rdma = pltpu.make_async_remote_copy(
                    src_ref=vcast.at[c],
                    dst_ref=vrx.at[my],
                    send_sem=v_send_sems.at[c],
                    recv_sem=v_recv_sems.at[my],
                    device_id=(c,),
                    device_id_type=pl.DeviceIdType.MESH,
                )
                v_rdma.start()
                kv_sends.append(k_rdma)
                kv_sends.append(v_rdma)

        lk = pltpu.make_async_copy(kcast.at[my], kbuf, local_sems.at[0])
        lk.start()
        lv = pltpu.make_async_copy(vcast.at[my], vbuf, local_sems.at[1])
        lv.start()

        wq = wq_ref[...].astype(BF16)
        for b in range(B):
            q = lax.dot_general(
                x_ref[b].astype(BF16), wq,
                (((1,), (0,)), ((), ())),
                preferred_element_type=F32)
            qbuf[b] = (q * 0.125).astype(BF16)

        qb_iota = lax.broadcasted_iota(jnp.int32, (SQ, KV_LOC), 0) // QBLK
        kb_loc = lax.broadcasted_iota(jnp.int32, (SQ, KV_LOC), 1) // QBLK

        def slot_update(src, first):
            kb = kb_loc + src * (KV_LOC // QBLK)
            allow = (qb_iota == kb) | (kb == 0) | (lax.rem(qb_iota + kb, 3) == 0)
            mask_j = jnp.where(allow, 0.0, -1e9).astype(F32)
            for b in range(B):
                for h in range(H_LOC):
                    q_bh = qbuf[b, :, pl.ds(h * DH, DH)]
                    k_bh = kbuf[b, :, pl.ds(h * DH, DH)]
                    s = lax.dot_general(
                        q_bh, k_bh, (((1,), (1,)), ((), ())),
                        preferred_element_type=F32) + mask_j
                    m_new = jnp.max(s, axis=-1, keepdims=True)
                    if not first:
                        m_old = m_ref[b, :, pl.ds(h, 1)]
                        m_new = jnp.maximum(m_old, m_new)
                    p = jnp.exp(s - m_new)
                    psum = jnp.sum(p, axis=-1, keepdims=True)
                    v_bh = vbuf[b, :, pl.ds(h * DH, DH)]
                    pv = lax.dot_general(
                        p.astype(BF16), v_bh, (((1,), (0,)), ((), ())),
                        preferred_element_type=F32)
                    if first:
                        l_new, acc_new = psum, pv
                    else:
                        corr = jnp.exp(m_old - m_new)
                        l_new = l_ref[b, :, pl.ds(h, 1)] * corr + psum
                        acc_new = acc_ref[b, :, pl.ds(h * DH, DH)] * corr + pv
                    m_ref[b, :, pl.ds(h, 1)] = m_new
                    l_ref[b, :, pl.ds(h, 1)] = l_new
                    acc_ref[b, :, pl.ds(h * DH, DH)] = acc_new

        lk.wait()
        lv.wait()
        slot_update(my, first=True)

        def slot_step(d, carry):
            src = lax.rem(my - d + N, N)
            pltpu.make_async_remote_copy(
                src_ref=kcast.at[0],
                dst_ref=krx.at[src],
                send_sem=k_send_sems.at[src],
                recv_sem=k_recv_sems.at[src],
                device_id=(src,),
                device_id_type=pl.DeviceIdType.MESH,
            ).wait_recv()
            pltpu.make_async_remote_copy(
                src_ref=vcast.at[0],
                dst_ref=vrx.at[src],
                send_sem=v_send_sems.at[src],
                recv_sem=v_recv_sems.at[src],
                device_id=(src,),
                device_id_type=pl.DeviceIdType.MESH,
            ).wait_recv()
            fk = pltpu.make_async_copy(krx.at[src], kbuf, local_sems.at[0])
            fk.start()
            fv = pltpu.make_async_copy(vrx.at[src], vbuf, local_sems.at[1])
            fv.start()
            fk.wait()
            fv.wait()
            slot_update(src, first=False)
            return carry

        lax.fori_loop(1, N, slot_step, jnp.int32(0))
        for rdma in kv_sends:
            rdma.wait_send()

        wo = wo_ref[...].astype(BF16)
        for b in range(B):
            acc_b = acc_ref[b].reshape(SQ, H_LOC, DH)
            l_b = l_ref[b].reshape(SQ, H_LOC, 1)
            ctx_b = (acc_b / l_b).reshape(SQ, HD).astype(BF16)
            partial_ref[b] = lax.dot_general(
                ctx_b, wo, (((1,), (0,)), ((), ())),
                preferred_element_type=F32).astype(BF16)

        rs_sends = []
        for d in range(1, N):
            peer = lax.rem(my + d, N)
            rdma = pltpu.make_async_remote_copy(
                src_ref=partial_ref.at[:, pl.ds(peer * CH, CH), :],
                dst_ref=rs_buf.at[my],
                send_sem=rs_send_sems.at[peer],
                recv_sem=rs_recv_sems.at[my],
                device_id=(peer,),
                device_id_type=pl.DeviceIdType.MESH,
            )
            rdma.start()
            rs_sends.append(rdma)
        red = partial_ref[:, pl.ds(my * CH, CH), :].astype(F32)
        for d in range(1, N):
            src = lax.rem(my - d + N, N)
            pltpu.make_async_remote_copy(
                src_ref=partial_ref.at[:, pl.ds(0, CH), :],
                dst_ref=rs_buf.at[src],
                send_sem=rs_send_sems.at[src],
                recv_sem=rs_recv_sems.at[src],
                device_id=(src,),
                device_id_type=pl.DeviceIdType.MESH,
            ).wait_recv()
            red = red + rs_buf[src].astype(F32)
        red_buf[...] = red.astype(BF16)
        out_ref[:, pl.ds(my * CH, CH), :] = red

        ag_sends = []
        for d in range(1, N):
            peer = lax.rem(my + d, N)
            rdma = pltpu.make_async_remote_copy(
                src_ref=red_buf,
                dst_ref=agbuf.at[:, pl.ds(my * CH, CH), :],
                send_sem=ag_send_sems.at[peer],
                recv_sem=ag_recv_sems.at[my],
                device_id=(peer,),
                device_id_type=pl.DeviceIdType.MESH,
            )
            rdma.start()
            ag_sends.append(rdma)
        for d in range(1, N):
            src = lax.rem(my - d + N, N)
            pltpu.make_async_remote_copy(
                src_ref=red_buf,
                dst_ref=agbuf.at[:, pl.ds(src * CH, CH), :],
                send_sem=ag_send_sems.at[src],
                recv_sem=ag_recv_sems.at[src],
                device_id=(src,),
                device_id_type=pl.DeviceIdType.MESH,
            ).wait_recv()
            out_ref[:, pl.ds(src * CH, CH), :] = (
                agbuf[:, pl.ds(src * CH, CH), :].astype(F32))
        for rdma in rs_sends + ag_sends:
            rdma.wait_send()

    vmem = pltpu.MemorySpace.VMEM
    hbm = pltpu.MemorySpace.HBM
    out = pl.pallas_call(
        body,
        out_shape=[
            jax.ShapeDtypeStruct((B, SQ, DM), F32),
            jax.ShapeDtypeStruct((N, B, KV_LOC, HD), BF16),
            jax.ShapeDtypeStruct((N, B, KV_LOC, HD), BF16),
        ],
        in_specs=[
            pl.BlockSpec(memory_space=vmem),
            pl.BlockSpec(memory_space=vmem),
            pl.BlockSpec(memory_space=hbm),
            pl.BlockSpec(memory_space=hbm),
            pl.BlockSpec(memory_space=vmem),
        ],
        out_specs=[
            pl.BlockSpec(memory_space=vmem),
            pl.BlockSpec(memory_space=hbm),
            pl.BlockSpec(memory_space=hbm),
        ],
        scratch_shapes=[
            vmem((N, B, KV_LOC, HD), BF16),
            vmem((N, B, KV_LOC, HD), BF16),
            vmem((B, KV_LOC, HD), BF16),
            vmem((B, KV_LOC, HD), BF16),
            vmem((KV_LOC, HCH, DH), F32),
            vmem((B, SQ, HD), BF16),
            vmem((B, SQ, H_LOC), F32),
            vmem((B, SQ, H_LOC), F32),
            vmem((B, SQ, HD), F32),
            vmem((B, SQ, DM), BF16),
            vmem((N, B, CH, DM), BF16),
            vmem((B, CH, DM), BF16),
            vmem((B, SQ, DM), BF16),
            pltpu.SemaphoreType.DMA((N,)),
            pltpu.SemaphoreType.DMA((N,)),
            pltpu.SemaphoreType.DMA((N,)),
            pltpu.SemaphoreType.DMA((N,)),
            pltpu.SemaphoreType.DMA((N,)),
            pltpu.SemaphoreType.DMA((N,)),
            pltpu.SemaphoreType.DMA((N,)),
            pltpu.SemaphoreType.DMA((N,)),
            pltpu.SemaphoreType.DMA((2,)),
        ],
        compiler_params=pltpu.CompilerParams(
            vmem_limit_bytes=100 * 1024 * 1024,
        ),
    )(x, Wq, K_ext, V_ext, Wo)
    return out[0]
